# baseline (speedup 1.0000x reference)
import functools

import jax
import jax.numpy as jnp
from jax import lax
from jax.experimental import pallas as pl
from jax.experimental.pallas import tpu as pltpu
from jax.experimental.pallas import tpu_sc as plsc

_NUM_CORES = 2
_NUM_SUBCORES = 16
_NUM_WORKERS = _NUM_CORES * _NUM_SUBCORES
_PIECE = 32768
_NBUF = 3


def kernel(x, table):
    seq_len = x.shape[1]
    dim = table.shape[1]
    total = seq_len * dim
    chunk = total // _NUM_WORKERS
    assert chunk * _NUM_WORKERS == total and chunk % 8 == 0
    trips = pl.cdiv(chunk, _PIECE)

    @functools.partial(
        pl.kernel,
        out_type=jax.ShapeDtypeStruct((total,), table.dtype),
        mesh=plsc.VectorSubcoreMesh(core_axis_name="c", subcore_axis_name="s"),
        scratch_types=(
            [pltpu.VMEM((_PIECE,), jnp.float32)] * _NBUF
            + [pltpu.SemaphoreType.DMA] * (2 * _NBUF)
        ),
    )
    def copy_k(table_hbm, out_hbm, *scratch):
        bufs = scratch[:_NBUF]
        in_sems = scratch[_NBUF : 2 * _NBUF]
        out_sems = scratch[2 * _NBUF :]
        wid = lax.axis_index("s") * _NUM_CORES + lax.axis_index("c")
        wbase = wid * chunk

        def piece_base(p):
            return jnp.minimum(wbase + p * _PIECE, wbase + chunk - _PIECE)

        def start_in(p):
            b = p % _NBUF
            return pltpu.async_copy(
                table_hbm.at[pl.ds(piece_base(p), _PIECE)],
                bufs[b],
                in_sems[b],
            )

        def start_out(p):
            b = p % _NBUF
            return pltpu.async_copy(
                bufs[b],
                out_hbm.at[pl.ds(piece_base(p), _PIECE)],
                out_sems[b],
            )

        in_h = [None] * trips
        out_h = [None] * trips
        out_waited = [False] * trips
        for b in range(min(_NBUF - 1, trips)):
            in_h[b] = start_in(b)
        for p in range(trips):
            in_h[p].wait()
            out_h[p] = start_out(p)
            nxt = p + _NBUF - 1
            if nxt < trips:
                if p >= 1:
                    out_h[p - 1].wait()
                    out_waited[p - 1] = True
                in_h[nxt] = start_in(nxt)
        for p in range(trips):
            if not out_waited[p]:
                out_h[p].wait()

    flat = copy_k(table.reshape(-1))
    return flat.reshape(seq_len, dim)


# speedup vs baseline: 1.0241x; 1.0241x over previous
import functools

import jax
import jax.numpy as jnp
from jax import lax
from jax.experimental import pallas as pl
from jax.experimental.pallas import tpu as pltpu
from jax.experimental.pallas import tpu_sc as plsc

_NUM_CORES = 2
_NUM_SUBCORES = 16
_NUM_WORKERS = _NUM_CORES * _NUM_SUBCORES
_PIECE = 16384
_NBUF = 6


def kernel(x, table):
    seq_len = x.shape[1]
    dim = table.shape[1]
    total = seq_len * dim
    chunk = total // _NUM_WORKERS
    assert chunk * _NUM_WORKERS == total and chunk % 8 == 0
    trips = pl.cdiv(chunk, _PIECE)

    @functools.partial(
        pl.kernel,
        out_type=jax.ShapeDtypeStruct((total,), table.dtype),
        mesh=plsc.VectorSubcoreMesh(core_axis_name="c", subcore_axis_name="s"),
        scratch_types=(
            [pltpu.VMEM((_PIECE,), jnp.float32)] * _NBUF
            + [pltpu.SemaphoreType.DMA] * (2 * _NBUF)
        ),
    )
    def copy_k(table_hbm, out_hbm, *scratch):
        bufs = scratch[:_NBUF]
        in_sems = scratch[_NBUF : 2 * _NBUF]
        out_sems = scratch[2 * _NBUF :]
        wid = lax.axis_index("s") * _NUM_CORES + lax.axis_index("c")
        wbase = wid * chunk

        def piece_base(p):
            return jnp.minimum(wbase + p * _PIECE, wbase + chunk - _PIECE)

        def start_in(p):
            b = p % _NBUF
            return pltpu.async_copy(
                table_hbm.at[pl.ds(piece_base(p), _PIECE)],
                bufs[b],
                in_sems[b],
            )

        def start_out(p):
            b = p % _NBUF
            return pltpu.async_copy(
                bufs[b],
                out_hbm.at[pl.ds(piece_base(p), _PIECE)],
                out_sems[b],
            )

        in_h = [None] * trips
        out_h = [None] * trips
        out_waited = [False] * trips
        for b in range(min(_NBUF - 1, trips)):
            in_h[b] = start_in(b)
        for p in range(trips):
            in_h[p].wait()
            out_h[p] = start_out(p)
            nxt = p + _NBUF - 1
            if nxt < trips:
                if p >= 1:
                    out_h[p - 1].wait()
                    out_waited[p - 1] = True
                in_h[nxt] = start_in(nxt)
        for p in range(trips):
            if not out_waited[p]:
                out_h[p].wait()

    flat = copy_k(table.reshape(-1))
    return flat.reshape(seq_len, dim)
